# baseline (speedup 1.0000x reference)
import jax
import jax.numpy as jnp
from jax import lax
from jax.experimental import pallas as pl
from jax.experimental.pallas import tpu as pltpu
from jax.experimental.pallas import tpu_sc as plsc

BATCH = 16384
NC, NS = 2, 16
NW = NC * NS
BPW = BATCH // NW


def _body(aid_hbm, atab_hbm, out_hbm, out_v):
    w = lax.axis_index("s") * NC + lax.axis_index("c")
    base = w * BPW
    for j in range(BPW // 128):
        pltpu.sync_copy(out_v.at[j], out_hbm.at[pl.ds(base + j * 128, 128)])


@jax.jit
def _run(author_ids, paper_ids, author_table, paper_table):
    mesh = plsc.VectorSubcoreMesh(core_axis_name="c", subcore_axis_name="s")
    return pl.kernel(
        _body,
        out_type=jax.ShapeDtypeStruct((BATCH,), jnp.float32),
        mesh=mesh,
        scratch_types=[
            pltpu.VMEM((BPW // 128, 128), jnp.float32),
        ],
    )(author_ids, paper_table.reshape(-1))


def kernel(author_ids, paper_ids, author_table, paper_table):
    return _run(author_ids, paper_ids, author_table, paper_table)


# speedup vs baseline: 1.5208x; 1.5208x over previous
import functools

import jax
import jax.numpy as jnp
from jax import lax
from jax.experimental import pallas as pl
from jax.experimental.pallas import tpu as pltpu
from jax.experimental.pallas import tpu_sc as plsc

DIM = 64
BATCH = 16384

NUM_CORES = 2
NUM_SUBCORES = 16
NUM_WORKERS = NUM_CORES * NUM_SUBCORES
B_PER_W = BATCH // NUM_WORKERS
C = 32
NCHUNK = B_PER_W // C


def _body(aid_hbm, pid_hbm, atab_hbm, ptab_hbm, out_hbm,
          aidx_v, pidx_v, abuf_v, pbuf_v, out_v,
          asem0, asem1, psem0, psem1):
    wid = lax.axis_index("s") * NUM_CORES + lax.axis_index("c")
    base = wid * B_PER_W

    for j in range(B_PER_W // 128):
        pltpu.sync_copy(aid_hbm.at[pl.ds(base + j * 128, 128)], aidx_v.at[j])
        pltpu.sync_copy(pid_hbm.at[pl.ds(base + j * 128, 128)], pidx_v.at[j])

    asems = [asem0, asem1]
    psems = [psem0, psem1]

    def fetch(g, bi):
        for grp in range(C // 16):
            pos0 = g * C + grp * 16
            avec = aidx_v[pos0 // 128, pl.ds(pos0 % 128, 16)]
            pvec = pidx_v[pos0 // 128, pl.ds(pos0 % 128, 16)]
            for rr in range(16):
                c = grp * 16 + rr
                pltpu.async_copy(atab_hbm.at[avec[rr]], abuf_v.at[bi, c], asems[bi])
                pltpu.async_copy(ptab_hbm.at[pvec[rr]], pbuf_v.at[bi, c], psems[bi])

    def drain(bi):
        pltpu.make_async_copy(atab_hbm.at[pl.ds(0, C)], abuf_v.at[bi], asems[bi]).wait()
        pltpu.make_async_copy(ptab_hbm.at[pl.ds(0, C)], pbuf_v.at[bi], psems[bi]).wait()

    lanes = lax.iota(jnp.int32, 16)
    masks = [(lanes & k) != 0 for k in (1, 2, 4, 8)]
    perms = [lanes ^ k for k in (1, 2, 4, 8)]

    def permute(v, idx):
        return v.at[idx].get(mode="promise_in_bounds")

    def merge(x, y, lvl):
        return jnp.where(masks[lvl], y, x) + permute(jnp.where(masks[lvl], x, y), perms[lvl])

    def compute(g, bi):
        for grp in range(C // 16):
            vs = []
            for rr in range(16):
                c = grp * 16 + rr
                acc = abuf_v[bi, c, pl.ds(0, 16)] * pbuf_v[bi, c, pl.ds(0, 16)]
                for k in range(1, DIM // 16):
                    acc = acc + (abuf_v[bi, c, pl.ds(k * 16, 16)]
                                 * pbuf_v[bi, c, pl.ds(k * 16, 16)])
                vs.append(acc)
            for lvl in range(4):
                vs = [merge(vs[2 * i], vs[2 * i + 1], lvl) for i in range(len(vs) // 2)]
            pos = g * C + grp * 16
            out_v[pos // 128, pl.ds(pos % 128, 16)] = vs[0]

    fetch(0, 0)

    def step(h, _):
        g = h * 2
        fetch(g + 1, 1)
        drain(0)
        compute(g, 0)

        @pl.when(h < NCHUNK // 2 - 1)
        def _():
            fetch(g + 2, 0)

        drain(1)
        compute(g + 1, 1)
        return 0

    lax.fori_loop(0, NCHUNK // 2, step, 0)

    for j in range(B_PER_W // 128):
        pltpu.sync_copy(out_v.at[j], out_hbm.at[pl.ds(base + j * 128, 128)])


@jax.jit
def _run(author_ids, paper_ids, author_table, paper_table):
    mesh = plsc.VectorSubcoreMesh(core_axis_name="c", subcore_axis_name="s")
    return pl.kernel(
        _body,
        out_type=jax.ShapeDtypeStruct((BATCH,), jnp.float32),
        mesh=mesh,
        scratch_types=[
            pltpu.VMEM((B_PER_W // 128, 128), jnp.int32),
            pltpu.VMEM((B_PER_W // 128, 128), jnp.int32),
            pltpu.VMEM((2, C, DIM), jnp.float32),
            pltpu.VMEM((2, C, DIM), jnp.float32),
            pltpu.VMEM((B_PER_W // 128, 128), jnp.float32),
            pltpu.SemaphoreType.DMA,
            pltpu.SemaphoreType.DMA,
            pltpu.SemaphoreType.DMA,
            pltpu.SemaphoreType.DMA,
        ],
    )(author_ids, paper_ids, author_table, paper_table)


def kernel(author_ids, paper_ids, author_table, paper_table):
    return _run(author_ids, paper_ids, author_table, paper_table)
